# baseline (speedup 1.0000x reference)
import functools

import jax
import jax.numpy as jnp
from jax import lax
from jax.experimental import pallas as pl
from jax.experimental.pallas import tpu as pltpu
from jax.experimental.pallas import tpu_sc as plsc


def _make_gather(V, D, B):
    info = plsc.get_sparse_core_info()
    NC, NS = info.num_cores, info.num_subcores
    NW = NC * NS
    assert B % NW == 0
    b_per_w = B // NW
    CH = 32
    NB = 3
    assert b_per_w % CH == 0
    n_ch = b_per_w // CH

    mesh = plsc.VectorSubcoreMesh(core_axis_name="c", subcore_axis_name="s")

    @functools.partial(
        pl.kernel,
        mesh=mesh,
        out_type=jax.ShapeDtypeStruct((B, D), jnp.float32),
        scratch_types=[
            pltpu.VMEM((b_per_w,), jnp.int32),
            pltpu.VMEM((NB, CH, D), jnp.float32),
        ]
        + [pltpu.SemaphoreType.DMA] * (2 * NB),
    )
    def gather_kernel(ids_hbm, table_hbm, out_hbm, idx_v, rows_v, *sems):
        sg = sems[:NB]
        so = sems[NB:]
        wid = lax.axis_index("s") * NC + lax.axis_index("c")
        base = wid * b_per_w
        pltpu.sync_copy(ids_hbm.at[pl.ds(base, b_per_w)], idx_v)

        def start_gather(c):
            return pltpu.async_copy(
                table_hbm.at[idx_v.at[pl.ds(c * CH, CH)]],
                rows_v.at[c % NB],
                sg[c % NB],
            )

        def start_out(c):
            return pltpu.async_copy(
                rows_v.at[c % NB],
                out_hbm.at[pl.ds(base + c * CH, CH)],
                so[c % NB],
            )

        gathers, outs = {}, {}
        fired = 0
        while fired < min(NB - 1, n_ch):
            gathers[fired] = start_gather(fired)
            fired += 1
        for c in range(n_ch):
            if fired < n_ch:
                prev_out = fired - NB
                if prev_out >= 0:
                    outs.pop(prev_out).wait()
                gathers[fired] = start_gather(fired)
                fired += 1
            gathers.pop(c).wait()
            outs[c] = start_out(c)
        for c in sorted(outs):
            outs.pop(c).wait()

    return gather_kernel


def kernel(position_ids, table):
    Bb, S = position_ids.shape
    V, D = table.shape
    B = Bb * S
    ids_flat = position_ids.reshape(B).astype(jnp.int32)
    out = _make_gather(V, D, B)(ids_flat, table)
    return out.reshape(Bb, S, D)


# speedup vs baseline: 1.0029x; 1.0029x over previous
import functools

import jax
import jax.numpy as jnp
from jax import lax
from jax.experimental import pallas as pl
from jax.experimental.pallas import tpu as pltpu
from jax.experimental.pallas import tpu_sc as plsc


def _make_gather(V, D, B):
    info = plsc.get_sparse_core_info()
    NC, NS = info.num_cores, info.num_subcores
    NW = NC * NS
    assert B % NW == 0
    b_per_w = B // NW
    CH = 16
    NB = 7
    assert b_per_w % CH == 0
    n_ch = b_per_w // CH

    mesh = plsc.VectorSubcoreMesh(core_axis_name="c", subcore_axis_name="s")

    @functools.partial(
        pl.kernel,
        mesh=mesh,
        out_type=jax.ShapeDtypeStruct((B, D), jnp.float32),
        scratch_types=[
            pltpu.VMEM((b_per_w,), jnp.int32),
            pltpu.VMEM((NB, CH, D), jnp.float32),
        ]
        + [pltpu.SemaphoreType.DMA] * (2 * NB),
    )
    def gather_kernel(ids_hbm, table_hbm, out_hbm, idx_v, rows_v, *sems):
        sg = sems[:NB]
        so = sems[NB:]
        wid = lax.axis_index("s") * NC + lax.axis_index("c")
        base = wid * b_per_w
        pltpu.sync_copy(ids_hbm.at[pl.ds(base, b_per_w)], idx_v)

        def start_gather(c):
            return pltpu.async_copy(
                table_hbm.at[idx_v.at[pl.ds(c * CH, CH)]],
                rows_v.at[c % NB],
                sg[c % NB],
            )

        def start_out(c):
            return pltpu.async_copy(
                rows_v.at[c % NB],
                out_hbm.at[pl.ds(base + c * CH, CH)],
                so[c % NB],
            )

        gathers, outs = {}, {}
        fired = 0
        while fired < min(NB - 1, n_ch):
            gathers[fired] = start_gather(fired)
            fired += 1
        for c in range(n_ch):
            if fired < n_ch:
                prev_out = fired - NB
                if prev_out >= 0:
                    outs.pop(prev_out).wait()
                gathers[fired] = start_gather(fired)
                fired += 1
            gathers.pop(c).wait()
            outs[c] = start_out(c)
        for c in sorted(outs):
            outs.pop(c).wait()

    return gather_kernel


def kernel(position_ids, table):
    Bb, S = position_ids.shape
    V, D = table.shape
    B = Bb * S
    ids_flat = position_ids.reshape(B).astype(jnp.int32)
    out = _make_gather(V, D, B)(ids_flat, table)
    return out.reshape(Bb, S, D)
